# baseline (speedup 1.0000x reference)
import functools

import jax
import jax.numpy as jnp
from jax import lax
from jax.experimental import pallas as pl
from jax.experimental.pallas import tpu as pltpu
from jax.experimental.pallas import tpu_sc as plsc

N = 10000
NP = 10240
E = 320000
NC = 2
NS = 16
EPC = E // NC
EPT = EPC // NS
B = 80
NB = EPT // B
ROWS_PT = NP // NS
DW = 16
R = 1000
GRID = N // R


SB = 640
NBAT = EPC // SB
CH = SB // 16
SEL = 128
FLUSH_AT = SEL - 16
ACC_R = ROWS_PT + 16


def _make_scan_agg(F, ones):
    def body(*refs):
        if ones:
            (dst_hbm, out_hbm, sbd, sel_d, rows_v, acc_v, sem) = refs
        else:
            (y_hbm, src_hbm, dst_hbm, out_hbm,
             sbs, sbd, sel_s, sel_d, rows_v, acc_v, sem) = refs
        cid = lax.axis_index("c")
        sid = lax.axis_index("s")
        lo = sid * ROWS_PT

        def zrow(i, c):
            for j in range(F // 16):
                acc_v[i, pl.ds(j * 16, 16)] = jnp.zeros((16,), jnp.float32)
            return c
        lax.fori_loop(0, ACC_R, zrow, 0)

        if ones:
            def orow(i, c):
                for j in range(F // 16):
                    rows_v[i, pl.ds(j * 16, 16)] = jnp.full((16,), 1.0,
                                                            jnp.float32)
                return c
            lax.fori_loop(0, SEL, orow, 0)
        else:
            for j in range(SEL // 16):
                sel_s[pl.ds(j * 16, 16)] = jnp.zeros((16,), jnp.int32)
        for j in range(SEL // 16):
            sel_d[pl.ds(j * 16, 16)] = jnp.full((16,), ROWS_PT, jnp.int32)

        col = lax.iota(jnp.int32, 16)

        def flush():
            if not ones:
                pltpu.async_copy(y_hbm.at[sel_s], rows_v, sem).wait()

            def acc_chunk(q, c):
                rbase = q * 16
                for e16 in range(16):
                    row_i = plsc.load_gather(
                        sel_d, [jnp.full((16,), 0, jnp.int32) + rbase + e16])
                    for j in range(F // 16):
                        plsc.addupdate_scatter(
                            acc_v, [row_i, col + j * 16],
                            rows_v[rbase + e16, pl.ds(j * 16, 16)])
                return c
            lax.fori_loop(0, SEL // 16, acc_chunk, 0)
            for j in range(SEL // 16):
                sel_d[pl.ds(j * 16, 16)] = jnp.full((16,), ROWS_PT,
                                                    jnp.int32)

        base_e = cid * EPC

        def batch(i, ptr):
            base = pl.multiple_of(base_e + i * SB, 8)
            if not ones:
                pltpu.sync_copy(src_hbm.at[pl.ds(base, SB)], sbs)
            pltpu.sync_copy(dst_hbm.at[pl.ds(base, SB)], sbd)

            def chunk(k, ptr):
                @pl.when(ptr > FLUSH_AT)
                def _():
                    flush()
                ptr = jnp.where(ptr > FLUSH_AT, 0, ptr)
                off = k * 16
                d16 = sbd[pl.ds(off, 16)]
                m = (d16 >= lo) & (d16 < lo + ROWS_PT)
                mi = m.astype(jnp.int32)
                cs = lax.cumsum(mi)
                pos = (ptr + cs) - 1
                if not ones:
                    s16 = sbs[pl.ds(off, 16)]
                    plsc.store_scatter(sel_s, [pos], s16, mask=m)
                plsc.store_scatter(sel_d, [pos], d16 - lo, mask=m)
                return ptr + jnp.sum(mi)
            return lax.fori_loop(0, CH, chunk, ptr)

        ptr = lax.fori_loop(0, NBAT, batch, jnp.int32(0))

        @pl.when(ptr > 0)
        def _():
            flush()

        pltpu.sync_copy(acc_v.at[pl.ds(0, ROWS_PT)],
                        out_hbm.at[cid, pl.ds(pl.multiple_of(lo, 128),
                                              ROWS_PT)])

    scratch = []
    if not ones:
        scratch.append(pltpu.VMEM((SB,), jnp.int32))
    scratch.append(pltpu.VMEM((SB,), jnp.int32))
    if not ones:
        scratch.append(pltpu.VMEM((SEL,), jnp.int32))
    scratch.append(pltpu.VMEM((SEL,), jnp.int32))
    scratch.append(pltpu.VMEM((SEL, F), jnp.float32))
    scratch.append(pltpu.VMEM((ACC_R, F), jnp.float32))
    scratch.append(pltpu.SemaphoreType.DMA)

    return pl.kernel(
        body,
        out_type=jax.ShapeDtypeStruct((NC, NP, F), jnp.float32),
        mesh=plsc.VectorSubcoreMesh(core_axis_name="c", subcore_axis_name="s"),
        scratch_types=scratch,
        compiler_params=pltpu.CompilerParams(needs_layout_passes=False),
    )


@functools.cache
def _deg_call():
    return _make_scan_agg(DW, ones=True)


@functools.cache
def _make_agg(F):
    return _make_scan_agg(F, ones=False)


def _dinv(degp_ref):
    deg = degp_ref[0][:, 0:1] + degp_ref[1][:, 0:1] + 1.0
    return lax.rsqrt(deg)


def _mm1_body(x_ref, w_ref, degp_ref, y_ref):
    y_ref[...] = jnp.dot(x_ref[...], w_ref[...],
                         preferred_element_type=jnp.float32) * _dinv(degp_ref)


def _bn_stats_body(aggp_ref, y_ref, degp_ref, b_ref, h_ref, sums_ref):
    i = pl.program_id(0)
    p = aggp_ref[0] + aggp_ref[1] + y_ref[...]
    h = jnp.maximum(p * _dinv(degp_ref) + b_ref[...], 0.0)
    h_ref[...] = h
    s = jnp.sum(h, axis=0, keepdims=True)
    q = jnp.sum(h * h, axis=0, keepdims=True)
    contrib = jnp.concatenate(
        [s, q, jnp.zeros((6, h.shape[1]), jnp.float32)], axis=0)

    @pl.when(i == 0)
    def _():
        sums_ref[...] = contrib

    @pl.when(i > 0)
    def _():
        sums_ref[...] += contrib


def _bn_fold(sums_ref, g_ref, b_ref):
    m = sums_ref[0:1, :] * (1.0 / N)
    q = sums_ref[1:2, :] * (1.0 / N)
    var = q - m * m
    s = g_ref[...] * lax.rsqrt(var + 1e-5)
    t = b_ref[...] - m * s
    return s, t


def _mm2_body(h_ref, sums_ref, degp_ref, g_ref, b_ref, w_ref, y2_ref):
    s, t = _bn_fold(sums_ref, g_ref, b_ref)
    hn = h_ref[...] * s + t
    y2 = jnp.dot(hn, w_ref[...],
                 preferred_element_type=jnp.float32) * _dinv(degp_ref)
    y2_ref[...] = jnp.concatenate(
        [y2, jnp.zeros((y2.shape[0], 64), jnp.float32)], axis=1)


def _out_body(h_ref, sums_ref, g_ref, b_ref, w_ref, lb_ref, out_ref):
    s, t = _bn_fold(sums_ref, g_ref, b_ref)
    hn = h_ref[...] * s + t
    out_ref[...] = jnp.dot(hn, w_ref[...],
                           preferred_element_type=jnp.float32) + lb_ref[...]


def _rows_spec(f):
    return pl.BlockSpec((R, f), lambda i: (i, 0))


def _degp_spec():
    return pl.BlockSpec((NC, R, DW), lambda i: (0, i, 0))


def _full_spec(shape):
    return pl.BlockSpec(shape, lambda i: tuple(0 for _ in shape))


def _mm1(x, W1, degp):
    return pl.pallas_call(
        _mm1_body,
        grid=(GRID,),
        in_specs=[_rows_spec(128), _full_spec((128, 128)), _degp_spec()],
        out_specs=_rows_spec(128),
        out_shape=jax.ShapeDtypeStruct((N, 128), jnp.float32),
    )(x, W1, degp)


def _bn_stats(aggp, y, degp, b, F):
    return pl.pallas_call(
        _bn_stats_body,
        grid=(GRID,),
        in_specs=[pl.BlockSpec((NC, R, F), lambda i: (0, i, 0)),
                  _rows_spec(F), _degp_spec(), _full_spec((1, F))],
        out_specs=[_rows_spec(F), _full_spec((8, F))],
        out_shape=[jax.ShapeDtypeStruct((N, F), jnp.float32),
                   jax.ShapeDtypeStruct((8, F), jnp.float32)],
    )(aggp, y, degp, b)


def _mm2(h, sums, degp, g, b, W2):
    return pl.pallas_call(
        _mm2_body,
        grid=(GRID,),
        in_specs=[_rows_spec(128), _full_spec((8, 128)), _degp_spec(),
                  _full_spec((1, 128)), _full_spec((1, 128)),
                  _full_spec((128, 64))],
        out_specs=_rows_spec(128),
        out_shape=jax.ShapeDtypeStruct((N, 128), jnp.float32),
    )(h, sums, degp, g, b, W2)


def _outk(h2, sums2, g, b, lin_W, lin_b):
    return pl.pallas_call(
        _out_body,
        grid=(GRID,),
        in_specs=[_rows_spec(128), _full_spec((8, 128)),
                  _full_spec((1, 128)), _full_spec((1, 128)),
                  _full_spec((128, 16)), _full_spec((1, 16))],
        out_specs=_rows_spec(16),
        out_shape=jax.ShapeDtypeStruct((N, 16), jnp.float32),
    )(h2, sums2, g, b, lin_W, lin_b)


def kernel(x, edge_index, W1, b1, bn1_g, bn1_b, W2, b2, bn2_g, bn2_b,
           lin_W, lin_b):
    src = edge_index[0]
    dst = edge_index[1]

    degp = _deg_call()(dst)
    y1 = _mm1(x, W1, degp)
    aggp1 = _make_agg(128)(y1, src, dst)
    h1, sums1 = _bn_stats(aggp1, y1, degp, b1.reshape(1, -1), 128)
    y2 = _mm2(h1, sums1, degp, bn1_g.reshape(1, -1), bn1_b.reshape(1, -1), W2)
    aggp2 = _make_agg(128)(y2, src, dst)
    b2p = jnp.pad(b2, (0, 64)).reshape(1, -1)
    g2p = jnp.pad(bn2_g, (0, 64), constant_values=1.0).reshape(1, -1)
    bb2p = jnp.pad(bn2_b, (0, 64)).reshape(1, -1)
    lwp = jnp.pad(lin_W, ((0, 64), (0, 0)))
    h2, sums2 = _bn_stats(aggp2, y2, degp, b2p, 128)
    return _outk(h2, sums2, g2p, bb2p, lwp, lin_b.reshape(1, -1))
